# Initial kernel scaffold; baseline (speedup 1.0000x reference)
import jax
import jax.numpy as jnp
from jax.experimental import pallas as pl


def kernel(img_feats, depth):
    raise NotImplementedError("write your pallas kernel here")


# speedup vs baseline: 1.2136x; 1.2136x over previous
import math

import jax
import jax.numpy as jnp
from jax import lax
from jax.experimental import pallas as pl
from jax.experimental.pallas import tpu as pltpu

_BS = 16
_FC = 128
_N = 16384
_MAP_HW = 101
_CELLS = _MAP_HW * _MAP_HW
_CELLS_PAD = 10208
_SENT = _CELLS
_EPS = -1e16
_MAP_SCALE = 0.1
_MAX_DEPTH = 10.0
_HFOV = math.radians(90.0)
_W = 512
_FX = _W / 2 * (1.0 / math.tan(_HFOV / 2))
_CX = _W / 2


def _index_body(dsub_ref, lin_ref):
    z = dsub_ref[...] * _MAX_DEPTH
    valid = jnp.abs(z) > 0.8
    zf = jnp.round(-(z / _MAP_SCALE) + (_MAP_HW - 1))
    j = lax.broadcasted_iota(jnp.int32, (_BS, 128, 128), 2).astype(jnp.float32)
    x = j * 4.0 + 2.0
    xx = (x - _CX) / _FX
    xf = jnp.round((xx * z) / _MAP_SCALE + (_MAP_HW - 1) / 2)
    r0 = zf.astype(jnp.int32)
    c0 = xf.astype(jnp.int32)
    invalid = (
        (r0 >= _MAP_HW) | (c0 >= _MAP_HW) | (r0 < 0) | (c0 < 0)
        | jnp.logical_not(valid)
    )
    lin_ref[...] = jnp.where(invalid, _SENT, r0 * _MAP_HW + c0)


def _scatter_body(lin_ref, feats_ref, out_ref):
    out_ref[...] = jnp.full(out_ref.shape, _EPS, jnp.float32)

    def step(p, carry):
        r = lin_ref[0, 0, p]
        row = feats_ref[0, pl.ds(p, 1), :]
        cur = out_ref[0, pl.ds(r, 1), :]
        out_ref[0, pl.ds(r, 1), :] = jnp.maximum(cur, row)
        return carry

    lax.fori_loop(0, _N, step, 0)
    cleaned = out_ref[...]
    out_ref[...] = jnp.where(cleaned == _EPS, 0.0, cleaned)


def kernel(img_feats, depth):
    dsub = depth[:, 0, 2::4, 2::4]
    lin = pl.pallas_call(
        _index_body,
        out_shape=jax.ShapeDtypeStruct((_BS, 128, 128), jnp.int32),
    )(dsub)
    lin3 = lin.reshape(_BS, 1, _N)
    feats_t = img_feats.reshape(_BS, _FC, _N).transpose(0, 2, 1)
    out = pl.pallas_call(
        _scatter_body,
        grid=(_BS,),
        in_specs=[
            pl.BlockSpec((1, 1, _N), lambda b: (b, 0, 0),
                         memory_space=pltpu.SMEM),
            pl.BlockSpec((1, _N, _FC), lambda b: (b, 0, 0)),
        ],
        out_specs=pl.BlockSpec((1, _CELLS_PAD, _FC), lambda b: (b, 0, 0)),
        out_shape=jax.ShapeDtypeStruct((_BS, _CELLS_PAD, _FC), jnp.float32),
    )(lin3, feats_t)
    out = out[:, :_CELLS, :].transpose(0, 2, 1)
    return out.reshape(_BS, _FC, _MAP_HW, _MAP_HW)
